# Initial kernel scaffold; baseline (speedup 1.0000x reference)
import jax
import jax.numpy as jnp
from jax.experimental import pallas as pl


def kernel(x, edge_index, batch, params):
    raise NotImplementedError("write your pallas kernel here")


# speedup vs baseline: 3.2257x; 3.2257x over previous
import jax
import jax.numpy as jnp
from jax import lax
from jax.experimental import pallas as pl
from jax.experimental.pallas import tpu as pltpu
from jax.experimental.pallas import tpu_sc as plsc

N = 10000
E = 320000
D = 128
H = 256
H2 = 512
L = 3
G = 64
BN_EPS = 1e-5

HHALF = 128
NS = 16
EPT = E // NS
CH = 80
NCHUNK = EPT // CH
NROWCH = N // CH

RB = 1000
NBLK = N // RB


def _sc_agg_body(h_lo, h_hi, src_hbm, dst_hbm, agg_lo, agg_hi,
                 idx_v, rows_v, sem, acc):
    c = lax.axis_index("c")
    s = lax.axis_index("s")

    if True:
        def zrow(i, carry):
            for k in range(HHALF // 16):
                rows_v[i, pl.ds(k * 16, 16)] = jnp.zeros((16,), jnp.float32)
            return carry
        lax.fori_loop(0, CH, zrow, 0)

        def zchunk(k, carry):
            idx = s + k * NS

            @pl.when(idx < NROWCH)
            def _():
                pltpu.sync_copy(rows_v, acc.at[pl.ds(idx * CH, CH)])
            return carry
        lax.fori_loop(0, (NROWCH + NS - 1) // NS, zchunk, 0)
        plsc.subcore_barrier()

        def edges(h_ref):
            base = s * EPT

            def chunk(j, carry):
                b = base + j * CH
                pltpu.sync_copy(src_hbm.at[pl.ds(b, CH)], idx_v.at[0])
                pltpu.sync_copy(dst_hbm.at[pl.ds(b, CH)], idx_v.at[1])
                pltpu.async_copy(h_ref.at[idx_v.at[0]], rows_v, sem).wait()
                pltpu.sync_copy(rows_v, acc.at[idx_v.at[1]], add=True)
                return carry
            lax.fori_loop(0, NCHUNK, chunk, 0)

        @pl.when(c == 0)
        def _():
            edges(h_lo)

        @pl.when(c == 1)
        def _():
            edges(h_hi)

        plsc.subcore_barrier()

        def writeback(out_ref):
            def wchunk(k, carry):
                idx = s + k * NS

                @pl.when(idx < NROWCH)
                def _():
                    pltpu.sync_copy(acc.at[pl.ds(idx * CH, CH)],
                                    out_ref.at[pl.ds(idx * CH, CH)])
                return carry
            lax.fori_loop(0, (NROWCH + NS - 1) // NS, wchunk, 0)

        @pl.when(c == 0)
        def _():
            writeback(agg_lo)

        @pl.when(c == 1)
        def _():
            writeback(agg_hi)


def _sc_agg(h_lo, h_hi, src, dst):
    mesh = plsc.VectorSubcoreMesh(core_axis_name="c", subcore_axis_name="s")
    return pl.kernel(
        _sc_agg_body,
        out_type=(jax.ShapeDtypeStruct((N, HHALF), jnp.float32),
                  jax.ShapeDtypeStruct((N, HHALF), jnp.float32)),
        mesh=mesh,
        scratch_types=(pltpu.VMEM((2, CH), jnp.int32),
                       pltpu.VMEM((CH, HHALF), jnp.float32),
                       pltpu.SemaphoreType.DMA,
                       pltpu.VMEM_SHARED((N, HHALF), jnp.float32)),
    )(h_lo, h_hi, src, dst)


def _stats_update(z, s_ref, q_ref, width):
    i = pl.program_id(0)
    mu_b = jnp.mean(z, axis=0, keepdims=True)
    zc = z - mu_b
    m2_b = jnp.sum(zc * zc, axis=0, keepdims=True)

    @pl.when(i == 0)
    def _():
        s_ref[...] = jnp.broadcast_to(mu_b, (8, width))
        q_ref[...] = jnp.broadcast_to(m2_b, (8, width))

    @pl.when(i > 0)
    def _():
        nprev = (i * RB).astype(jnp.float32)
        nnew = nprev + RB
        mu_run = s_ref[0:1, :]
        delta = mu_b - mu_run
        s_ref[...] = jnp.broadcast_to(
            mu_run + delta * (RB / nnew), (8, width))
        q_ref[...] = jnp.broadcast_to(
            q_ref[0:1, :] + m2_b + delta * delta * (nprev * RB / nnew),
            (8, width))


def _bn_apply(z, s_ref, q_ref, g_ref, be_ref):
    mean = s_ref[0:1, :]
    var = q_ref[0:1, :] * (1.0 / N)
    return g_ref[...] * (z - mean) / jnp.sqrt(var + BN_EPS) + be_ref[...]

def _mlp_in(x, w, b):
    def body(x_ref, w_ref, b_ref, lo_ref, hi_ref):
        z = jnp.dot(x_ref[...], w_ref[...], preferred_element_type=jnp.float32)
        z = jnp.maximum(z + b_ref[...], 0.0)
        lo_ref[...] = z[:, :HHALF]
        hi_ref[...] = z[:, HHALF:]

    return pl.pallas_call(
        body,
        grid=(NBLK,),
        in_specs=[pl.BlockSpec((RB, D), lambda i: (i, 0)),
                  pl.BlockSpec((D, H), lambda i: (0, 0)),
                  pl.BlockSpec((1, H), lambda i: (0, 0))],
        out_specs=[pl.BlockSpec((RB, HHALF), lambda i: (i, 0)),
                   pl.BlockSpec((RB, HHALF), lambda i: (i, 0))],
        out_shape=[jax.ShapeDtypeStruct((N, HHALF), jnp.float32)] * 2,
    )(x, w, b)


def _k1(h_lo, h_hi, agg_lo, agg_hi, w1, b1, scale):
    def body(hlo_ref, hhi_ref, alo_ref, ahi_ref, w1_ref, b1_ref, sc_ref,
             z_ref, s_ref, q_ref):
        sc = sc_ref[0, 0]
        u = jnp.concatenate(
            [sc * hlo_ref[...] + alo_ref[...],
             sc * hhi_ref[...] + ahi_ref[...]], axis=1)
        z = jnp.dot(u, w1_ref[...], preferred_element_type=jnp.float32)
        z = z + b1_ref[...]
        z_ref[...] = z
        _stats_update(z, s_ref, q_ref, H2)

    return pl.pallas_call(
        body,
        grid=(NBLK,),
        in_specs=[pl.BlockSpec((RB, HHALF), lambda i: (i, 0)),
                  pl.BlockSpec((RB, HHALF), lambda i: (i, 0)),
                  pl.BlockSpec((RB, HHALF), lambda i: (i, 0)),
                  pl.BlockSpec((RB, HHALF), lambda i: (i, 0)),
                  pl.BlockSpec((H, H2), lambda i: (0, 0)),
                  pl.BlockSpec((1, H2), lambda i: (0, 0)),
                  pl.BlockSpec(memory_space=pltpu.SMEM)],
        out_specs=[pl.BlockSpec((RB, H2), lambda i: (i, 0)),
                   pl.BlockSpec((8, H2), lambda i: (0, 0)),
                   pl.BlockSpec((8, H2), lambda i: (0, 0))],
        out_shape=[jax.ShapeDtypeStruct((N, H2), jnp.float32),
                   jax.ShapeDtypeStruct((8, H2), jnp.float32),
                   jax.ShapeDtypeStruct((8, H2), jnp.float32)],
    )(h_lo, h_hi, agg_lo, agg_hi, w1, b1, scale)


def _k2(z1, s1, q1, g1, be1, w2, b2):
    def body(z_ref, s1_ref, q1_ref, g_ref, be_ref, w2_ref, b2_ref,
             z2_ref, s_ref, q_ref):
        a = jnp.maximum(_bn_apply(z_ref[...], s1_ref, q1_ref, g_ref, be_ref),
                        0.0)
        z2 = jnp.dot(a, w2_ref[...], preferred_element_type=jnp.float32)
        z2 = z2 + b2_ref[...]
        z2_ref[...] = z2
        _stats_update(z2, s_ref, q_ref, H)

    return pl.pallas_call(
        body,
        grid=(NBLK,),
        in_specs=[pl.BlockSpec((RB, H2), lambda i: (i, 0)),
                  pl.BlockSpec((8, H2), lambda i: (0, 0)),
                  pl.BlockSpec((8, H2), lambda i: (0, 0)),
                  pl.BlockSpec((1, H2), lambda i: (0, 0)),
                  pl.BlockSpec((1, H2), lambda i: (0, 0)),
                  pl.BlockSpec((H2, H), lambda i: (0, 0)),
                  pl.BlockSpec((1, H), lambda i: (0, 0))],
        out_specs=[pl.BlockSpec((RB, H), lambda i: (i, 0)),
                   pl.BlockSpec((8, H), lambda i: (0, 0)),
                   pl.BlockSpec((8, H), lambda i: (0, 0))],
        out_shape=[jax.ShapeDtypeStruct((N, H), jnp.float32),
                   jax.ShapeDtypeStruct((8, H), jnp.float32),
                   jax.ShapeDtypeStruct((8, H), jnp.float32)],
    )(z1, s1, q1, g1, be1, w2, b2)


def _k3(z2, s2, q2, g, b):
    def body(z_ref, s_ref, q_ref, g_ref, be_ref, lo_ref, hi_ref):
        hcur = jnp.maximum(_bn_apply(z_ref[...], s_ref, q_ref, g_ref, be_ref),
                           0.0)
        lo_ref[...] = hcur[:, :HHALF]
        hi_ref[...] = hcur[:, HHALF:]

    return pl.pallas_call(
        body,
        grid=(NBLK,),
        in_specs=[pl.BlockSpec((RB, H), lambda i: (i, 0)),
                  pl.BlockSpec((8, H), lambda i: (0, 0)),
                  pl.BlockSpec((8, H), lambda i: (0, 0)),
                  pl.BlockSpec((1, H), lambda i: (0, 0)),
                  pl.BlockSpec((1, H), lambda i: (0, 0))],
        out_specs=[pl.BlockSpec((RB, HHALF), lambda i: (i, 0)),
                   pl.BlockSpec((RB, HHALF), lambda i: (i, 0))],
        out_shape=[jax.ShapeDtypeStruct((N, HHALF), jnp.float32)] * 2,
    )(z2, s2, q2, g, b)


def _k3_last(z2, s2, q2, g, b, bt, wl1, bl1, wl2p, bl2p):
    def body(z_ref, s_ref, q_ref, g_ref, be_ref, bt_ref,
             wl1_ref, bl1_ref, wl2_ref, bl2_ref, out_ref, pool_ref):
        hcur = jnp.maximum(_bn_apply(z_ref[...], s_ref, q_ref, g_ref, be_ref),
                           0.0)
        bcol = bt_ref[...][:, 0:1]
        oh = (bcol == lax.broadcasted_iota(jnp.int32, (1, G), 1)
              ).astype(jnp.float32)
        part = lax.dot_general(oh, hcur, (((0,), (0,)), ((), ())),
                               preferred_element_type=jnp.float32)
        i = pl.program_id(0)

        @pl.when(i == 0)
        def _():
            pool_ref[...] = jnp.zeros_like(pool_ref)
        pool_ref[...] += part

        @pl.when(i == NBLK - 1)
        def _():
            o = jnp.dot(pool_ref[...], wl1_ref[...],
                        preferred_element_type=jnp.float32) + bl1_ref[...]
            o = jnp.maximum(o, 0.0)
            out_ref[...] = jnp.dot(o, wl2_ref[...],
                                   preferred_element_type=jnp.float32) \
                + bl2_ref[...]

    return pl.pallas_call(
        body,
        grid=(NBLK,),
        in_specs=[pl.BlockSpec((RB, H), lambda i: (i, 0)),
                  pl.BlockSpec((8, H), lambda i: (0, 0)),
                  pl.BlockSpec((8, H), lambda i: (0, 0)),
                  pl.BlockSpec((1, H), lambda i: (0, 0)),
                  pl.BlockSpec((1, H), lambda i: (0, 0)),
                  pl.BlockSpec((RB, 8), lambda i: (i, 0)),
                  pl.BlockSpec((H, H), lambda i: (0, 0)),
                  pl.BlockSpec((1, H), lambda i: (0, 0)),
                  pl.BlockSpec((H, HHALF), lambda i: (0, 0)),
                  pl.BlockSpec((1, HHALF), lambda i: (0, 0))],
        out_specs=pl.BlockSpec((G, HHALF), lambda i: (0, 0)),
        out_shape=jax.ShapeDtypeStruct((G, HHALF), jnp.float32),
        scratch_shapes=[pltpu.VMEM((G, H), jnp.float32)],
    )(z2, s2, q2, g, b, bt, wl1, bl1, wl2p, bl2p)


def kernel(x, edge_index, batch, params):
    p = params
    src = edge_index[0]
    dst = edge_index[1]
    bt = jnp.broadcast_to(batch[:, None], (N, 8))
    wl2p = jnp.pad(p['W_l2'], ((0, 0), (0, HHALF - 1)))
    bl2p = jnp.pad(p['b_l2'].reshape(1, 1), ((0, 0), (0, HHALF - 1)))

    h_lo, h_hi = _mlp_in(x, p['W_in'], p['b_in'].reshape(1, H))
    out = None
    for l in range(L):
        agg_lo, agg_hi = _sc_agg(h_lo, h_hi, src, dst)
        scale = (1.0 + p['eps'][l]).reshape(1, 1)
        z1, s1, q1 = _k1(h_lo, h_hi, agg_lo, agg_hi,
                         p[f'c{l}_W1'], p[f'c{l}_b1'].reshape(1, H2), scale)
        z2, s2, q2 = _k2(z1, s1, q1, p[f'c{l}_g1'].reshape(1, H2),
                         p[f'c{l}_be1'].reshape(1, H2),
                         p[f'c{l}_W2'], p[f'c{l}_b2'].reshape(1, H))
        if l < L - 1:
            h_lo, h_hi = _k3(z2, s2, q2, p[f'bn{l}_g'].reshape(1, H),
                             p[f'bn{l}_b'].reshape(1, H))
        else:
            out = _k3_last(z2, s2, q2, p[f'bn{l}_g'].reshape(1, H),
                           p[f'bn{l}_b'].reshape(1, H), bt,
                           p['W_l1'], p['b_l1'].reshape(1, H),
                           wl2p, bl2p)
    return out[:, 0:1]
